# baseline (speedup 1.0000x reference)
import functools

import jax
import jax.numpy as jnp
from jax import lax
from jax.experimental import pallas as pl
from jax.experimental.pallas import tpu as pltpu
from jax.experimental.pallas import tpu_sc as plsc

N = 100000
D = 128
B = 256
NEG = -1e30
NCG = D // 16

R1 = 2000
NB1 = N // R1


def _gate_body(feat_ref, wg_ref, g_ref, k_ref, ksc_ref):
    i = pl.program_id(0)

    @pl.when(i == 0)
    def _init():
        ksc_ref[0, 0] = NEG

    gfull = lax.dot_general(feat_ref[...], wg_ref[...], (((1,), (0,)), ((), ())),
                            preferred_element_type=jnp.float32)
    g_ref[0, 0, :] = gfull[:, 0]
    ksc_ref[0, 0] = jnp.maximum(ksc_ref[0, 0], jnp.max(gfull[:, :8]))

    @pl.when(i == NB1 - 1)
    def _fin():
        k_ref[...] = jnp.full((1, 128), ksc_ref[0, 0], jnp.float32)


@jax.jit
def _gate_pass(feat, W_gate):
    return pl.pallas_call(
        _gate_body,
        grid=(NB1,),
        in_specs=[
            pl.BlockSpec((R1, D), lambda i: (i, 0)),
            pl.BlockSpec((D, D), lambda i: (0, 0)),
        ],
        out_specs=[
            pl.BlockSpec((1, 1, R1), lambda i: (i, 0, 0)),
            pl.BlockSpec((1, 128), lambda i: (0, 0)),
        ],
        out_shape=[
            jax.ShapeDtypeStruct((NB1, 1, R1), jnp.float32),
            jax.ShapeDtypeStruct((1, 128), jnp.float32),
        ],
        scratch_shapes=[pltpu.SMEM((1, 1), jnp.float32)],
    )(feat, jnp.tile(W_gate, (1, D)))


NW = 32
CR = 160
NCH = N // CR
MAXK = -(-NCH // NW)
FULL = NCH - (MAXK - 1) * NW
NV = CR // 16


def _sc_body(feat_hbm, g_hbm, seg_hbm, k_hbm, s_out, d_out,
             k_v, sacc, dacc, f0, f1, g0, g1, c0, c1, sem0, sem1):
    wid = lax.axis_index("s") * 2 + lax.axis_index("c")
    nch = jnp.where(wid < FULL, MAXK, MAXK - 1)

    pltpu.sync_copy(k_hbm.at[pl.ds(0, 16)], k_v)

    zv = jnp.zeros((16,), jnp.float32)
    lane = lax.broadcasted_iota(jnp.int32, (16,), 0)

    def _zero_s(r, c):
        sacc[pl.ds(r * 16, 16)] = zv
        return c
    lax.fori_loop(0, B * D // 16, _zero_s, 0)

    def _zero_d(r, c):
        dacc[pl.ds(r * 16, 16)] = zv
        return c
    lax.fori_loop(0, B, _zero_d, 0)

    bufs = ((f0, g0, c0, sem0), (f1, g1, c1, sem1))

    def _start(k, buf):
        f_v, g_v, c_v, sem = buf
        base = (wid + k * NW) * CR
        pltpu.async_copy(feat_hbm.at[pl.ds(base, CR), :], f_v, sem)
        pltpu.async_copy(g_hbm.at[pl.ds(base, CR)], g_v, sem)
        pltpu.async_copy(seg_hbm.at[pl.ds(base, CR)], c_v, sem)

    def _wait(buf):
        f_v, g_v, c_v, sem = buf
        pltpu.make_async_copy(feat_hbm.at[pl.ds(0, CR), :], f_v, sem).wait()
        pltpu.make_async_copy(g_hbm.at[pl.ds(0, CR)], g_v, sem).wait()
        pltpu.make_async_copy(seg_hbm.at[pl.ds(0, CR)], c_v, sem).wait()

    def _process(buf):
        f_v, g_v, c_v, sem = buf
        kv = k_v[...]

        def _half(r0, seg16, e16):
            s_first = seg16[0]
            s_last = seg16[15]
            uniform = s_first == s_last

            @pl.when(uniform)
            def _uni():
                na = [zv] * NCG
                for r in range(16):
                    er = e16[r]
                    for cg in range(NCG):
                        na[cg] = na[cg] + er * f_v[r0 + r, pl.ds(cg * 16, 16)]
                for cg in range(NCG):
                    sl = pl.ds(s_first * D + cg * 16, 16)
                    sacc[sl] = sacc[sl] + na[cg]
                dsl = pl.ds(s_first * 16, 16)
                dacc[dsl] = dacc[dsl] + e16

            @pl.when(jnp.logical_not(uniform))
            def _bnd():
                for r in range(16):
                    sr = seg16[r]
                    er = e16[r]
                    for cg in range(NCG):
                        sl = pl.ds(sr * D + cg * 16, 16)
                        sacc[sl] = sacc[sl] + er * f_v[r0 + r, pl.ds(cg * 16, 16)]
                    dsl = pl.ds(sr * 16, 16)
                    dacc[dsl] = dacc[dsl] + jnp.where(lane == r, e16, zv)

        def _vec(v, c):
            r0a = v * 32
            r0b = r0a + 16
            sla = pl.ds(r0a, 16)
            slb = pl.ds(r0b, 16)
            sega = c_v[sla]
            segb = c_v[slb]
            ea = jnp.exp(g_v[sla] - kv)
            eb = jnp.exp(g_v[slb] - kv)
            both_uni = (sega[0] == segb[15])

            @pl.when(both_uni)
            def _fused():
                na = [zv] * NCG
                nb = [zv] * NCG
                for r in range(16):
                    era = ea[r]
                    erb = eb[r]
                    for cg in range(NCG):
                        na[cg] = na[cg] + era * f_v[r0a + r, pl.ds(cg * 16, 16)]
                        nb[cg] = nb[cg] + erb * f_v[r0b + r, pl.ds(cg * 16, 16)]
                sf = sega[0]
                for cg in range(NCG):
                    sl = pl.ds(sf * D + cg * 16, 16)
                    sacc[sl] = sacc[sl] + (na[cg] + nb[cg])
                dsl = pl.ds(sf * 16, 16)
                dacc[dsl] = dacc[dsl] + (ea + eb)

            @pl.when(jnp.logical_not(both_uni))
            def _split():
                _half(r0a, sega, ea)
                _half(r0b, segb, eb)

            return c

        lax.fori_loop(0, NV // 2, _vec, 0)

    _start(0, bufs[0])

    def _pair(j, c):
        k0 = 2 * j
        k1 = 2 * j + 1

        @pl.when(k1 < nch)
        def _s1():
            _start(k1, bufs[1])

        @pl.when(k0 < nch)
        def _p0():
            _wait(bufs[0])
            _process(bufs[0])

        @pl.when(k1 + 1 < nch)
        def _s2():
            _start(k1 + 1, bufs[0])

        @pl.when(k1 < nch)
        def _p1():
            _wait(bufs[1])
            _process(bufs[1])

        return c

    lax.fori_loop(0, MAXK // 2, _pair, 0)

    pltpu.sync_copy(sacc, s_out.at[wid])
    pltpu.sync_copy(dacc, d_out.at[wid])


@jax.jit
def _sc_pass(feat, g, seg, k):
    mesh = plsc.VectorSubcoreMesh(core_axis_name="c", subcore_axis_name="s")
    kfn = functools.partial(
        pl.kernel,
        mesh=mesh,
        out_type=[
            jax.ShapeDtypeStruct((NW, B * D), jnp.float32),
            jax.ShapeDtypeStruct((NW, B * 16), jnp.float32),
        ],
        scratch_types=[
            pltpu.VMEM((16,), jnp.float32),
            pltpu.VMEM((B * D,), jnp.float32),
            pltpu.VMEM((B * 16,), jnp.float32),
            pltpu.VMEM((CR, D), jnp.float32),
            pltpu.VMEM((CR, D), jnp.float32),
            pltpu.VMEM((CR,), jnp.float32),
            pltpu.VMEM((CR,), jnp.float32),
            pltpu.VMEM((CR,), jnp.int32),
            pltpu.VMEM((CR,), jnp.int32),
            pltpu.SemaphoreType.DMA,
            pltpu.SemaphoreType.DMA,
        ],
    )(_sc_body)
    return kfn(feat, g, seg, k)


def _epi_body(s_ref, d_ref, wf_ref, bf_ref, out_ref):
    s = jnp.sum(s_ref[...], axis=0)
    d = jnp.sum(d_ref[...], axis=(0, 2))
    nonempty = d > 0.0
    a = s / jnp.where(nonempty, d, 1.0)[:, None]
    out = lax.dot_general(a, wf_ref[...], (((1,), (0,)), ((), ())),
                          preferred_element_type=jnp.float32)
    out_ref[...] = out + jnp.where(nonempty, 1.0, 0.0)[:, None] * bf_ref[0, :][None, :]


@jax.jit
def _epilogue(s_part, d_part, W_feat, bf_row):
    return pl.pallas_call(
        _epi_body,
        out_shape=jax.ShapeDtypeStruct((B, D), jnp.float32),
    )(s_part, d_part, W_feat, bf_row)


def kernel(feat, segment_ids, W_gate, b_gate, W_feat, b_feat):
    seg = segment_ids.astype(jnp.int32)
    bf_row = b_feat.reshape(1, D)
    g3, k2 = _gate_pass(feat, W_gate)
    s_part, d_part = _sc_pass(feat, g3.reshape(N), seg, k2.reshape(128))
    return _epilogue(s_part.reshape(NW, B, D), d_part.reshape(NW, B, 16),
                     W_feat, bf_row)


# speedup vs baseline: 1.1758x; 1.1758x over previous
import functools

import jax
import jax.numpy as jnp
from jax import lax
from jax.experimental import pallas as pl
from jax.experimental.pallas import tpu as pltpu
from jax.experimental.pallas import tpu_sc as plsc

N = 100000
D = 128
B = 256
NEG = -1e30
NCG = D // 16

R1 = 2000
NB1 = N // R1


def _gate_body(feat_ref, wg_ref, g_ref, k_ref, ksc_ref):
    i = pl.program_id(0)

    @pl.when(i == 0)
    def _init():
        ksc_ref[0, 0] = NEG

    g = jnp.sum(feat_ref[...] * wg_ref[0, :][None, :], axis=1)
    g_ref[0, 0, :] = g
    ksc_ref[0, 0] = jnp.maximum(ksc_ref[0, 0], jnp.max(g))

    @pl.when(i == NB1 - 1)
    def _fin():
        k_ref[...] = jnp.full((1, 128), ksc_ref[0, 0], jnp.float32)


@jax.jit
def _gate_pass(feat, W_gate):
    return pl.pallas_call(
        _gate_body,
        grid=(NB1,),
        in_specs=[
            pl.BlockSpec((R1, D), lambda i: (i, 0)),
            pl.BlockSpec((1, D), lambda i: (0, 0)),
        ],
        out_specs=[
            pl.BlockSpec((1, 1, R1), lambda i: (i, 0, 0)),
            pl.BlockSpec((1, 128), lambda i: (0, 0)),
        ],
        out_shape=[
            jax.ShapeDtypeStruct((NB1, 1, R1), jnp.float32),
            jax.ShapeDtypeStruct((1, 128), jnp.float32),
        ],
        scratch_shapes=[pltpu.SMEM((1, 1), jnp.float32)],
    )(feat, W_gate.reshape(1, D))


NW = 32
CR = 160
NCH = N // CR
MAXK = -(-NCH // NW)
FULL = NCH - (MAXK - 1) * NW
NV = CR // 16


def _sc_body(feat_hbm, g_hbm, seg_hbm, k_hbm, s_out, d_out,
             k_v, sacc, dacc, f0, f1, g0, g1, c0, c1, sem0, sem1):
    wid = lax.axis_index("s") * 2 + lax.axis_index("c")
    nch = jnp.where(wid < FULL, MAXK, MAXK - 1)

    pltpu.sync_copy(k_hbm.at[pl.ds(0, 16)], k_v)

    zv = jnp.zeros((16,), jnp.float32)
    lane = lax.broadcasted_iota(jnp.int32, (16,), 0)

    def _zero_s(r, c):
        sacc[pl.ds(r * 16, 16)] = zv
        return c
    lax.fori_loop(0, B * D // 16, _zero_s, 0)

    def _zero_d(r, c):
        dacc[pl.ds(r * 16, 16)] = zv
        return c
    lax.fori_loop(0, B, _zero_d, 0)

    bufs = ((f0, g0, c0, sem0), (f1, g1, c1, sem1))

    def _start(k, buf):
        f_v, g_v, c_v, sem = buf
        base = (wid + k * NW) * CR
        pltpu.async_copy(feat_hbm.at[pl.ds(base, CR), :], f_v, sem)
        pltpu.async_copy(g_hbm.at[pl.ds(base, CR)], g_v, sem)
        pltpu.async_copy(seg_hbm.at[pl.ds(base, CR)], c_v, sem)

    def _wait(buf):
        f_v, g_v, c_v, sem = buf
        pltpu.make_async_copy(feat_hbm.at[pl.ds(0, CR), :], f_v, sem).wait()
        pltpu.make_async_copy(g_hbm.at[pl.ds(0, CR)], g_v, sem).wait()
        pltpu.make_async_copy(seg_hbm.at[pl.ds(0, CR)], c_v, sem).wait()

    def _flush(seg_scalar, accs, accd_v):
        for cg in range(NCG):
            sl = pl.ds(seg_scalar * D + cg * 16, 16)
            sacc[sl] = sacc[sl] + accs[cg]
        dsl = pl.ds(seg_scalar * 16, 16)
        dacc[dsl] = dacc[dsl] + accd_v

    def _process(buf):
        f_v, g_v, c_v, sem = buf
        kv = k_v[...]

        def _vec(v, carry):
            accs = carry[:NCG]
            accd = carry[NCG]
            cur = carry[NCG + 1]
            r0 = v * 16
            sl16 = pl.ds(r0, 16)
            seg16 = c_v[sl16]
            e16 = jnp.exp(g_v[sl16] - kv)
            s_first = seg16[0]
            s_last = seg16[15]
            uniform = s_first == s_last

            @pl.when((cur >= 0) & ((s_first != cur) | jnp.logical_not(uniform)))
            def _fl():
                _flush(cur, accs, accd)

            @pl.when(jnp.logical_not(uniform))
            def _bnd():
                for r in range(16):
                    sr = seg16[r]
                    er = e16[r]
                    for cg in range(NCG):
                        sl = pl.ds(sr * D + cg * 16, 16)
                        sacc[sl] = sacc[sl] + er * f_v[r0 + r, pl.ds(cg * 16, 16)]
                    dsl = pl.ds(sr * 16, 16)
                    dacc[dsl] = dacc[dsl] + jnp.where(lane == r, e16, zv)

            u = jnp.where(uniform, 1.0, 0.0)
            keep = jnp.where(uniform & (s_first == cur), 1.0, 0.0)
            e16s = u * e16
            na = [keep * accs[cg] for cg in range(NCG)]
            for r in range(16):
                er = e16s[r]
                for cg in range(NCG):
                    na[cg] = na[cg] + er * f_v[r0 + r, pl.ds(cg * 16, 16)]
            new_accd = keep * accd + e16s
            new_cur = jnp.where(uniform, s_first, jnp.int32(-1))
            return tuple(na) + (new_accd, new_cur)

        init = (zv,) * NCG + (zv, jnp.int32(-1))
        fin = lax.fori_loop(0, NV, _vec, init)

        @pl.when(fin[NCG + 1] >= 0)
        def _last():
            _flush(fin[NCG + 1], fin[:NCG], fin[NCG])

    _start(0, bufs[0])

    def _pair(j, c):
        k0 = 2 * j
        k1 = 2 * j + 1

        @pl.when(k1 < nch)
        def _s1():
            _start(k1, bufs[1])

        @pl.when(k0 < nch)
        def _p0():
            _wait(bufs[0])
            _process(bufs[0])

        @pl.when(k1 + 1 < nch)
        def _s2():
            _start(k1 + 1, bufs[0])

        @pl.when(k1 < nch)
        def _p1():
            _wait(bufs[1])
            _process(bufs[1])

        return c

    lax.fori_loop(0, MAXK // 2, _pair, 0)

    pltpu.sync_copy(sacc, s_out.at[wid])
    pltpu.sync_copy(dacc, d_out.at[wid])


@jax.jit
def _sc_pass(feat, g, seg, k):
    mesh = plsc.VectorSubcoreMesh(core_axis_name="c", subcore_axis_name="s")
    kfn = functools.partial(
        pl.kernel,
        mesh=mesh,
        out_type=[
            jax.ShapeDtypeStruct((NW, B * D), jnp.float32),
            jax.ShapeDtypeStruct((NW, B * 16), jnp.float32),
        ],
        scratch_types=[
            pltpu.VMEM((16,), jnp.float32),
            pltpu.VMEM((B * D,), jnp.float32),
            pltpu.VMEM((B * 16,), jnp.float32),
            pltpu.VMEM((CR, D), jnp.float32),
            pltpu.VMEM((CR, D), jnp.float32),
            pltpu.VMEM((CR,), jnp.float32),
            pltpu.VMEM((CR,), jnp.float32),
            pltpu.VMEM((CR,), jnp.int32),
            pltpu.VMEM((CR,), jnp.int32),
            pltpu.SemaphoreType.DMA,
            pltpu.SemaphoreType.DMA,
        ],
    )(_sc_body)
    return kfn(feat, g, seg, k)


def _epi_body(s_ref, d_ref, wf_ref, bf_ref, out_ref):
    s = jnp.sum(s_ref[...], axis=0)
    d = jnp.sum(d_ref[...], axis=(0, 2))
    nonempty = d > 0.0
    a = s / jnp.where(nonempty, d, 1.0)[:, None]
    out = lax.dot_general(a, wf_ref[...], (((1,), (0,)), ((), ())),
                          preferred_element_type=jnp.float32)
    out_ref[...] = out + jnp.where(nonempty, 1.0, 0.0)[:, None] * bf_ref[0, :][None, :]


@jax.jit
def _epilogue(s_part, d_part, W_feat, bf_row):
    return pl.pallas_call(
        _epi_body,
        out_shape=jax.ShapeDtypeStruct((B, D), jnp.float32),
    )(s_part, d_part, W_feat, bf_row)


def kernel(feat, segment_ids, W_gate, b_gate, W_feat, b_feat):
    seg = segment_ids.astype(jnp.int32)
    bf_row = b_feat.reshape(1, D)
    g3, k2 = _gate_pass(feat, W_gate)
    s_part, d_part = _sc_pass(feat, g3.reshape(N), seg, k2.reshape(128))
    return _epilogue(s_part.reshape(NW, B, D), d_part.reshape(NW, B, 16),
                     W_feat, bf_row)


# speedup vs baseline: 1.3516x; 1.1495x over previous
import functools

import jax
import jax.numpy as jnp
from jax import lax
from jax.experimental import pallas as pl
from jax.experimental.pallas import tpu as pltpu
from jax.experimental.pallas import tpu_sc as plsc

N = 100000
D = 128
B = 256
NEG = -1e30
NCG = D // 16

R1 = 2000
NB1 = N // R1


def _gate_body(feat_ref, wg_ref, g_ref, k_ref, ksc_ref):
    i = pl.program_id(0)

    @pl.when(i == 0)
    def _init():
        ksc_ref[0, 0] = NEG

    g = jnp.sum(feat_ref[...] * wg_ref[0, :][None, :], axis=1)
    g_ref[0, 0, :] = g
    ksc_ref[0, 0] = jnp.maximum(ksc_ref[0, 0], jnp.max(g))

    @pl.when(i == NB1 - 1)
    def _fin():
        k_ref[...] = jnp.full((1, 128), ksc_ref[0, 0], jnp.float32)


@jax.jit
def _gate_pass(feat, W_gate):
    return pl.pallas_call(
        _gate_body,
        grid=(NB1,),
        in_specs=[
            pl.BlockSpec((R1, D), lambda i: (i, 0)),
            pl.BlockSpec((1, D), lambda i: (0, 0)),
        ],
        out_specs=[
            pl.BlockSpec((1, 1, R1), lambda i: (i, 0, 0)),
            pl.BlockSpec((1, 128), lambda i: (0, 0)),
        ],
        out_shape=[
            jax.ShapeDtypeStruct((NB1, 1, R1), jnp.float32),
            jax.ShapeDtypeStruct((1, 128), jnp.float32),
        ],
        scratch_shapes=[pltpu.SMEM((1, 1), jnp.float32)],
    )(feat, W_gate.reshape(1, D))


NW = 32
CR = 160
NCH = N // CR
MAXK = -(-NCH // NW)
FULL = NCH - (MAXK - 1) * NW
NV = CR // 16


def _sc_body(feat_hbm, g_hbm, seg_hbm, k_hbm, s_out, d_out,
             k_v, sacc, dacc, f0, f1, g0, g1, c0, c1, sem0, sem1):
    wid = lax.axis_index("s") * 2 + lax.axis_index("c")
    nch = jnp.where(wid < FULL, MAXK, MAXK - 1)

    pltpu.sync_copy(k_hbm.at[pl.ds(0, 16)], k_v)

    zv = jnp.zeros((16,), jnp.float32)
    lane = lax.broadcasted_iota(jnp.int32, (16,), 0)

    def _zero_s(r, c):
        for cg in range(NCG):
            sacc[r, pl.ds(cg * 16, 16)] = zv
        dacc[r, pl.ds(0, 16)] = zv
        return c
    lax.fori_loop(0, B, _zero_s, 0)

    bufs = ((f0, g0, c0, sem0), (f1, g1, c1, sem1))

    def _start(k, buf):
        f_v, g_v, c_v, sem = buf
        base = (wid + k * NW) * CR
        pltpu.async_copy(feat_hbm.at[pl.ds(base, CR), :], f_v, sem)
        pltpu.async_copy(g_hbm.at[pl.ds(base, CR)], g_v, sem)
        pltpu.async_copy(seg_hbm.at[pl.ds(base, CR)], c_v, sem)

    def _wait(buf):
        f_v, g_v, c_v, sem = buf
        pltpu.make_async_copy(feat_hbm.at[pl.ds(0, CR), :], f_v, sem).wait()
        pltpu.make_async_copy(g_hbm.at[pl.ds(0, CR)], g_v, sem).wait()
        pltpu.make_async_copy(seg_hbm.at[pl.ds(0, CR)], c_v, sem).wait()

    def _flush(seg_scalar, accs, accd_v):
        for cg in range(NCG):
            sl = pl.ds(cg * 16, 16)
            sacc[seg_scalar, sl] = sacc[seg_scalar, sl] + accs[cg]
        d16 = pl.ds(0, 16)
        dacc[seg_scalar, d16] = dacc[seg_scalar, d16] + accd_v

    def _process(buf):
        f_v, g_v, c_v, sem = buf
        kv = k_v[...]

        def _vec(v, carry):
            accs = carry[:NCG]
            accd = carry[NCG]
            cur = carry[NCG + 1]
            r0 = v * 16
            sl16 = pl.ds(r0, 16)
            seg16 = c_v[sl16]
            e16 = jnp.exp(g_v[sl16] - kv)
            s_first = seg16[0]
            s_last = seg16[15]
            uniform = s_first == s_last

            @pl.when((cur >= 0) & ((s_first != cur) | jnp.logical_not(uniform)))
            def _fl():
                _flush(cur, accs, accd)

            @pl.when(jnp.logical_not(uniform))
            def _bnd():
                d16 = pl.ds(0, 16)
                for r in range(16):
                    sr = seg16[r]
                    er = e16[r]
                    for cg in range(NCG):
                        sl = pl.ds(cg * 16, 16)
                        sacc[sr, sl] = sacc[sr, sl] + er * f_v[r0 + r, sl]
                    dacc[sr, d16] = dacc[sr, d16] + jnp.where(lane == r, e16, zv)

            u = jnp.where(uniform, 1.0, 0.0)
            keep = jnp.where(uniform & (s_first == cur), 1.0, 0.0)
            e16s = u * e16
            na = [keep * accs[cg] for cg in range(NCG)]
            for r in range(16):
                er = e16s[r]
                for cg in range(NCG):
                    na[cg] = na[cg] + er * f_v[r0 + r, pl.ds(cg * 16, 16)]
            new_accd = keep * accd + e16s
            new_cur = jnp.where(uniform, s_first, jnp.int32(-1))
            return tuple(na) + (new_accd, new_cur)

        init = (zv,) * NCG + (zv, jnp.int32(-1))
        fin = lax.fori_loop(0, NV, _vec, init)

        @pl.when(fin[NCG + 1] >= 0)
        def _last():
            _flush(fin[NCG + 1], fin[:NCG], fin[NCG])

    _start(0, bufs[0])

    def _pair(j, c):
        k0 = 2 * j
        k1 = 2 * j + 1

        @pl.when(k1 < nch)
        def _s1():
            _start(k1, bufs[1])

        @pl.when(k0 < nch)
        def _p0():
            _wait(bufs[0])
            _process(bufs[0])

        @pl.when(k1 + 1 < nch)
        def _s2():
            _start(k1 + 1, bufs[0])

        @pl.when(k1 < nch)
        def _p1():
            _wait(bufs[1])
            _process(bufs[1])

        return c

    lax.fori_loop(0, MAXK // 2, _pair, 0)

    pltpu.sync_copy(sacc, s_out.at[wid])
    pltpu.sync_copy(dacc, d_out.at[wid])


@jax.jit
def _sc_pass(feat, g, seg, k):
    mesh = plsc.VectorSubcoreMesh(core_axis_name="c", subcore_axis_name="s")
    kfn = functools.partial(
        pl.kernel,
        mesh=mesh,
        out_type=[
            jax.ShapeDtypeStruct((NW, B, D), jnp.float32),
            jax.ShapeDtypeStruct((NW, B, 16), jnp.float32),
        ],
        scratch_types=[
            pltpu.VMEM((16,), jnp.float32),
            pltpu.VMEM((B, D), jnp.float32),
            pltpu.VMEM((B, 16), jnp.float32),
            pltpu.VMEM((CR, D), jnp.float32),
            pltpu.VMEM((CR, D), jnp.float32),
            pltpu.VMEM((CR,), jnp.float32),
            pltpu.VMEM((CR,), jnp.float32),
            pltpu.VMEM((CR,), jnp.int32),
            pltpu.VMEM((CR,), jnp.int32),
            pltpu.SemaphoreType.DMA,
            pltpu.SemaphoreType.DMA,
        ],
    )(_sc_body)
    return kfn(feat, g, seg, k)


def _epi_body(s_ref, d_ref, wf_ref, bf_ref, out_ref):
    s = jnp.sum(s_ref[...], axis=0)
    d = jnp.sum(d_ref[...], axis=(0, 2))
    nonempty = d > 0.0
    a = s / jnp.where(nonempty, d, 1.0)[:, None]
    out = lax.dot_general(a, wf_ref[...], (((1,), (0,)), ((), ())),
                          preferred_element_type=jnp.float32)
    out_ref[...] = out + jnp.where(nonempty, 1.0, 0.0)[:, None] * bf_ref[0, :][None, :]


@jax.jit
def _epilogue(s_part, d_part, W_feat, bf_row):
    return pl.pallas_call(
        _epi_body,
        out_shape=jax.ShapeDtypeStruct((B, D), jnp.float32),
    )(s_part, d_part, W_feat, bf_row)


def kernel(feat, segment_ids, W_gate, b_gate, W_feat, b_feat):
    seg = segment_ids.astype(jnp.int32)
    bf_row = b_feat.reshape(1, D)
    g3, k2 = _gate_pass(feat, W_gate)
    s_part, d_part = _sc_pass(feat, g3.reshape(N), seg, k2.reshape(128))
    return _epilogue(s_part, d_part, W_feat, bf_row)


# speedup vs baseline: 1.4652x; 1.0840x over previous
import functools

import jax
import jax.numpy as jnp
from jax import lax
from jax.experimental import pallas as pl
from jax.experimental.pallas import tpu as pltpu
from jax.experimental.pallas import tpu_sc as plsc

N = 100000
D = 128
B = 256
NEG = -1e30
NCG = D // 16

R1 = 5000
NB1 = N // R1


def _gate_body(feat_ref, wg_ref, g_ref, k_ref, ksc_ref):
    i = pl.program_id(0)

    @pl.when(i == 0)
    def _init():
        ksc_ref[0, 0] = NEG

    g = jnp.sum(feat_ref[...] * wg_ref[0, :][None, :], axis=1)
    g_ref[0, 0, :] = g
    ksc_ref[0, 0] = jnp.maximum(ksc_ref[0, 0], jnp.max(g))

    @pl.when(i == NB1 - 1)
    def _fin():
        k_ref[...] = jnp.full((1, 128), ksc_ref[0, 0], jnp.float32)


@jax.jit
def _gate_pass(feat, W_gate):
    return pl.pallas_call(
        _gate_body,
        grid=(NB1,),
        in_specs=[
            pl.BlockSpec((R1, D), lambda i: (i, 0)),
            pl.BlockSpec((1, D), lambda i: (0, 0)),
        ],
        out_specs=[
            pl.BlockSpec((1, 1, R1), lambda i: (i, 0, 0)),
            pl.BlockSpec((1, 128), lambda i: (0, 0)),
        ],
        out_shape=[
            jax.ShapeDtypeStruct((NB1, 1, R1), jnp.float32),
            jax.ShapeDtypeStruct((1, 128), jnp.float32),
        ],
        scratch_shapes=[pltpu.SMEM((1, 1), jnp.float32)],
    )(feat, W_gate.reshape(1, D))


NW = 32
CR = 160
NCH = N // CR
MAXK = -(-NCH // NW)
FULL = NCH - (MAXK - 1) * NW
NV = CR // 16


def _sc_body(feat_hbm, g_hbm, seg_hbm, k_hbm, s_out, d_out,
             k_v, sacc, dacc, f0, f1, g0, g1, c0, c1, sem0, sem1):
    wid = lax.axis_index("s") * 2 + lax.axis_index("c")
    nch = jnp.where(wid < FULL, MAXK, MAXK - 1)

    pltpu.sync_copy(k_hbm.at[pl.ds(0, 16)], k_v)

    zv = jnp.zeros((16,), jnp.float32)
    lane = lax.broadcasted_iota(jnp.int32, (16,), 0)

    def _zero_s(r, c):
        for cg in range(NCG):
            sacc[r, pl.ds(cg * 16, 16)] = zv
        dacc[r, pl.ds(0, 16)] = zv
        return c
    lax.fori_loop(0, B, _zero_s, 0)

    bufs = ((f0, g0, c0, sem0), (f1, g1, c1, sem1))

    def _start(k, buf):
        f_v, g_v, c_v, sem = buf
        base = (wid + k * NW) * CR
        pltpu.async_copy(feat_hbm.at[pl.ds(base, CR), :], f_v, sem)
        pltpu.async_copy(g_hbm.at[pl.ds(base, CR)], g_v, sem)
        pltpu.async_copy(seg_hbm.at[pl.ds(base, CR)], c_v, sem)

    def _wait(buf):
        f_v, g_v, c_v, sem = buf
        pltpu.make_async_copy(feat_hbm.at[pl.ds(0, CR), :], f_v, sem).wait()
        pltpu.make_async_copy(g_hbm.at[pl.ds(0, CR)], g_v, sem).wait()
        pltpu.make_async_copy(seg_hbm.at[pl.ds(0, CR)], c_v, sem).wait()

    def _flush(seg_scalar, accs, accd_v):
        for cg in range(NCG):
            sl = pl.ds(cg * 16, 16)
            sacc[seg_scalar, sl] = sacc[seg_scalar, sl] + accs[cg]
        d16 = pl.ds(0, 16)
        dacc[seg_scalar, d16] = dacc[seg_scalar, d16] + accd_v

    def _process(buf):
        f_v, g_v, c_v, sem = buf
        kv = k_v[...]

        def _vec(v, carry):
            accs = carry[:NCG]
            accd = carry[NCG]
            cur = carry[NCG + 1]
            r0 = v * 16
            sl16 = pl.ds(r0, 16)
            seg16 = c_v[sl16]
            e16 = jnp.exp(g_v[sl16] - kv)
            s_first = seg16[0]
            s_last = seg16[15]
            uniform = s_first == s_last

            @pl.when((cur >= 0) & ((s_first != cur) | jnp.logical_not(uniform)))
            def _fl():
                _flush(cur, accs, accd)

            @pl.when(jnp.logical_not(uniform))
            def _bnd():
                d16 = pl.ds(0, 16)
                for r in range(16):
                    sr = seg16[r]
                    er = e16[r]
                    for cg in range(NCG):
                        sl = pl.ds(cg * 16, 16)
                        sacc[sr, sl] = sacc[sr, sl] + er * f_v[r0 + r, sl]
                    dacc[sr, d16] = dacc[sr, d16] + jnp.where(lane == r, e16, zv)

            u = jnp.where(uniform, 1.0, 0.0)
            keep = jnp.where(uniform & (s_first == cur), 1.0, 0.0)
            e16s = u * e16
            na = [keep * accs[cg] for cg in range(NCG)]
            for r in range(16):
                er = e16s[r]
                for cg in range(NCG):
                    na[cg] = na[cg] + er * f_v[r0 + r, pl.ds(cg * 16, 16)]
            new_accd = keep * accd + e16s
            new_cur = jnp.where(uniform, s_first, jnp.int32(-1))
            return tuple(na) + (new_accd, new_cur)

        init = (zv,) * NCG + (zv, jnp.int32(-1))
        fin = lax.fori_loop(0, NV, _vec, init)

        @pl.when(fin[NCG + 1] >= 0)
        def _last():
            _flush(fin[NCG + 1], fin[:NCG], fin[NCG])

    _start(0, bufs[0])

    def _pair(j, c):
        k0 = 2 * j
        k1 = 2 * j + 1

        @pl.when(k1 < nch)
        def _s1():
            _start(k1, bufs[1])

        @pl.when(k0 < nch)
        def _p0():
            _wait(bufs[0])
            _process(bufs[0])

        @pl.when(k1 + 1 < nch)
        def _s2():
            _start(k1 + 1, bufs[0])

        @pl.when(k1 < nch)
        def _p1():
            _wait(bufs[1])
            _process(bufs[1])

        return c

    lax.fori_loop(0, MAXK // 2, _pair, 0)

    pltpu.sync_copy(sacc, s_out.at[wid])
    pltpu.sync_copy(dacc, d_out.at[wid])


@jax.jit
def _sc_pass(feat, g, seg, k):
    mesh = plsc.VectorSubcoreMesh(core_axis_name="c", subcore_axis_name="s")
    kfn = functools.partial(
        pl.kernel,
        mesh=mesh,
        out_type=[
            jax.ShapeDtypeStruct((NW, B, D), jnp.float32),
            jax.ShapeDtypeStruct((NW, B, 16), jnp.float32),
        ],
        scratch_types=[
            pltpu.VMEM((16,), jnp.float32),
            pltpu.VMEM((B, D), jnp.float32),
            pltpu.VMEM((B, 16), jnp.float32),
            pltpu.VMEM((CR, D), jnp.float32),
            pltpu.VMEM((CR, D), jnp.float32),
            pltpu.VMEM((CR,), jnp.float32),
            pltpu.VMEM((CR,), jnp.float32),
            pltpu.VMEM((CR,), jnp.int32),
            pltpu.VMEM((CR,), jnp.int32),
            pltpu.SemaphoreType.DMA,
            pltpu.SemaphoreType.DMA,
        ],
    )(_sc_body)
    return kfn(feat, g, seg, k)


def _epi_body(s_ref, d_ref, wf_ref, bf_ref, out_ref):
    s = jnp.sum(s_ref[...], axis=0)
    d = jnp.sum(d_ref[...], axis=(0, 2))
    nonempty = d > 0.0
    a = s / jnp.where(nonempty, d, 1.0)[:, None]
    out = lax.dot_general(a, wf_ref[...], (((1,), (0,)), ((), ())),
                          preferred_element_type=jnp.float32)
    out_ref[...] = out + jnp.where(nonempty, 1.0, 0.0)[:, None] * bf_ref[0, :][None, :]


@jax.jit
def _epilogue(s_part, d_part, W_feat, bf_row):
    return pl.pallas_call(
        _epi_body,
        out_shape=jax.ShapeDtypeStruct((B, D), jnp.float32),
    )(s_part, d_part, W_feat, bf_row)


def kernel(feat, segment_ids, W_gate, b_gate, W_feat, b_feat):
    seg = segment_ids.astype(jnp.int32)
    bf_row = b_feat.reshape(1, D)
    g3, k2 = _gate_pass(feat, W_gate)
    s_part, d_part = _sc_pass(feat, g3.reshape(N), seg, k2.reshape(128))
    return _epilogue(s_part, d_part, W_feat, bf_row)


# speedup vs baseline: 1.4802x; 1.0102x over previous
import functools

import jax
import jax.numpy as jnp
from jax import lax
from jax.experimental import pallas as pl
from jax.experimental.pallas import tpu as pltpu
from jax.experimental.pallas import tpu_sc as plsc

N = 100000
D = 128
B = 256
NEG = -1e30
NCG = D // 16

R1 = 10000
NB1 = N // R1


def _gate_body(feat_ref, wg_ref, g_ref, k_ref, ksc_ref):
    i = pl.program_id(0)

    @pl.when(i == 0)
    def _init():
        ksc_ref[0, 0] = NEG

    g = jnp.sum(feat_ref[...] * wg_ref[0, :][None, :], axis=1)
    g_ref[0, 0, :] = g
    ksc_ref[0, 0] = jnp.maximum(ksc_ref[0, 0], jnp.max(g))

    @pl.when(i == NB1 - 1)
    def _fin():
        k_ref[...] = jnp.full((1, 128), ksc_ref[0, 0], jnp.float32)


@jax.jit
def _gate_pass(feat, W_gate):
    return pl.pallas_call(
        _gate_body,
        grid=(NB1,),
        in_specs=[
            pl.BlockSpec((R1, D), lambda i: (i, 0)),
            pl.BlockSpec((1, D), lambda i: (0, 0)),
        ],
        out_specs=[
            pl.BlockSpec((1, 1, R1), lambda i: (i, 0, 0)),
            pl.BlockSpec((1, 128), lambda i: (0, 0)),
        ],
        out_shape=[
            jax.ShapeDtypeStruct((NB1, 1, R1), jnp.float32),
            jax.ShapeDtypeStruct((1, 128), jnp.float32),
        ],
        scratch_shapes=[pltpu.SMEM((1, 1), jnp.float32)],
    )(feat, W_gate.reshape(1, D))


NW = 32
CR = 160
NCH = N // CR
MAXK = -(-NCH // NW)
FULL = NCH - (MAXK - 1) * NW
NV = CR // 16


def _sc_body(feat_hbm, g_hbm, seg_hbm, k_hbm, s_out, d_out,
             k_v, sacc, dacc, f0, f1, g0, g1, c0, c1, sem0, sem1):
    wid = lax.axis_index("s") * 2 + lax.axis_index("c")
    nch = jnp.where(wid < FULL, MAXK, MAXK - 1)

    pltpu.sync_copy(k_hbm.at[pl.ds(0, 16)], k_v)

    zv = jnp.zeros((16,), jnp.float32)
    lane = lax.broadcasted_iota(jnp.int32, (16,), 0)

    def _zero_s(r, c):
        for cg in range(NCG):
            sacc[r, pl.ds(cg * 16, 16)] = zv
        dacc[r, pl.ds(0, 16)] = zv
        return c
    lax.fori_loop(0, B, _zero_s, 0)

    bufs = ((f0, g0, c0, sem0), (f1, g1, c1, sem1))

    def _start(k, buf):
        f_v, g_v, c_v, sem = buf
        base = (wid + k * NW) * CR
        pltpu.async_copy(feat_hbm.at[pl.ds(base, CR), :], f_v, sem)
        pltpu.async_copy(g_hbm.at[pl.ds(base, CR)], g_v, sem)
        pltpu.async_copy(seg_hbm.at[pl.ds(base, CR)], c_v, sem)

    def _wait(buf):
        f_v, g_v, c_v, sem = buf
        pltpu.make_async_copy(feat_hbm.at[pl.ds(0, CR), :], f_v, sem).wait()
        pltpu.make_async_copy(g_hbm.at[pl.ds(0, CR)], g_v, sem).wait()
        pltpu.make_async_copy(seg_hbm.at[pl.ds(0, CR)], c_v, sem).wait()

    def _flush(seg_scalar, accs, accd_v):
        for cg in range(NCG):
            sl = pl.ds(cg * 16, 16)
            sacc[seg_scalar, sl] = sacc[seg_scalar, sl] + accs[cg]
        d16 = pl.ds(0, 16)
        dacc[seg_scalar, d16] = dacc[seg_scalar, d16] + accd_v

    def _process(buf):
        f_v, g_v, c_v, sem = buf
        kv = k_v[...]

        def _vec(v, carry):
            accs = carry[:NCG]
            accd = carry[NCG]
            cur = carry[NCG + 1]
            r0 = v * 16
            sl16 = pl.ds(r0, 16)
            seg16 = c_v[sl16]
            e16 = jnp.exp(g_v[sl16] - kv)
            s_first = seg16[0]
            s_last = seg16[15]
            uniform = s_first == s_last

            @pl.when((cur >= 0) & ((s_first != cur) | jnp.logical_not(uniform)))
            def _fl():
                _flush(cur, accs, accd)

            @pl.when(jnp.logical_not(uniform))
            def _bnd():
                d16 = pl.ds(0, 16)
                for r in range(16):
                    sr = seg16[r]
                    er = e16[r]
                    for cg in range(NCG):
                        sl = pl.ds(cg * 16, 16)
                        sacc[sr, sl] = sacc[sr, sl] + er * f_v[r0 + r, sl]
                    dacc[sr, d16] = dacc[sr, d16] + jnp.where(lane == r, e16, zv)

            u = jnp.where(uniform, 1.0, 0.0)
            keep = jnp.where(uniform & (s_first == cur), 1.0, 0.0)
            e16s = u * e16
            na = [keep * accs[cg] for cg in range(NCG)]
            for r in range(16):
                er = e16s[r]
                for cg in range(NCG):
                    na[cg] = na[cg] + er * f_v[r0 + r, pl.ds(cg * 16, 16)]
            new_accd = keep * accd + e16s
            new_cur = jnp.where(uniform, s_first, jnp.int32(-1))
            return tuple(na) + (new_accd, new_cur)

        init = (zv,) * NCG + (zv, jnp.int32(-1))
        fin = lax.fori_loop(0, NV, _vec, init)

        @pl.when(fin[NCG + 1] >= 0)
        def _last():
            _flush(fin[NCG + 1], fin[:NCG], fin[NCG])

    _start(0, bufs[0])

    def _pair(j, c):
        k0 = 2 * j
        k1 = 2 * j + 1

        @pl.when(k1 < nch)
        def _s1():
            _start(k1, bufs[1])

        @pl.when(k0 < nch)
        def _p0():
            _wait(bufs[0])
            _process(bufs[0])

        @pl.when(k1 + 1 < nch)
        def _s2():
            _start(k1 + 1, bufs[0])

        @pl.when(k1 < nch)
        def _p1():
            _wait(bufs[1])
            _process(bufs[1])

        return c

    lax.fori_loop(0, MAXK // 2, _pair, 0)

    pltpu.sync_copy(sacc, s_out.at[wid])
    pltpu.sync_copy(dacc, d_out.at[wid])


@jax.jit
def _sc_pass(feat, g, seg, k):
    mesh = plsc.VectorSubcoreMesh(core_axis_name="c", subcore_axis_name="s")
    kfn = functools.partial(
        pl.kernel,
        mesh=mesh,
        out_type=[
            jax.ShapeDtypeStruct((NW, B, D), jnp.float32),
            jax.ShapeDtypeStruct((NW, B, 16), jnp.float32),
        ],
        scratch_types=[
            pltpu.VMEM((16,), jnp.float32),
            pltpu.VMEM((B, D), jnp.float32),
            pltpu.VMEM((B, 16), jnp.float32),
            pltpu.VMEM((CR, D), jnp.float32),
            pltpu.VMEM((CR, D), jnp.float32),
            pltpu.VMEM((CR,), jnp.float32),
            pltpu.VMEM((CR,), jnp.float32),
            pltpu.VMEM((CR,), jnp.int32),
            pltpu.VMEM((CR,), jnp.int32),
            pltpu.SemaphoreType.DMA,
            pltpu.SemaphoreType.DMA,
        ],
    )(_sc_body)
    return kfn(feat, g, seg, k)


def _epi_body(s_ref, d_ref, wf_ref, bf_ref, out_ref):
    s = jnp.sum(s_ref[...], axis=0)
    d = jnp.sum(d_ref[...], axis=(0, 2))
    nonempty = d > 0.0
    a = s / jnp.where(nonempty, d, 1.0)[:, None]
    out = lax.dot_general(a, wf_ref[...], (((1,), (0,)), ((), ())),
                          preferred_element_type=jnp.float32)
    out_ref[...] = out + jnp.where(nonempty, 1.0, 0.0)[:, None] * bf_ref[0, :][None, :]


@jax.jit
def _epilogue(s_part, d_part, W_feat, bf_row):
    return pl.pallas_call(
        _epi_body,
        out_shape=jax.ShapeDtypeStruct((B, D), jnp.float32),
    )(s_part, d_part, W_feat, bf_row)


def kernel(feat, segment_ids, W_gate, b_gate, W_feat, b_feat):
    seg = segment_ids.astype(jnp.int32)
    bf_row = b_feat.reshape(1, D)
    g3, k2 = _gate_pass(feat, W_gate)
    s_part, d_part = _sc_pass(feat, g3.reshape(N), seg, k2.reshape(128))
    return _epilogue(s_part, d_part, W_feat, bf_row)
